# baseline (speedup 1.0000x reference)
import jax
import jax.numpy as jnp
from jax.experimental import pallas as pl
from jax.experimental.pallas import tpu as pltpu


def _copy_block(table_ref, out_ref):
    out_ref[...] = table_ref[...][None]


def kernel(token_ids, table):
    T_max, C = table.shape
    _, T = token_ids.shape
    BLOCK = 512
    grid = (T // BLOCK,)
    out = pl.pallas_call(
        _copy_block,
        grid=grid,
        in_specs=[pl.BlockSpec((BLOCK, C), lambda i: (i, 0))],
        out_specs=pl.BlockSpec((1, BLOCK, C), lambda i: (0, i, 0)),
        out_shape=jax.ShapeDtypeStruct((1, T, C), table.dtype),
    )(table)
    return out


# speedup vs baseline: 1.1748x; 1.1748x over previous
import jax
import jax.numpy as jnp
from jax.experimental import pallas as pl
from jax.experimental.pallas import tpu as pltpu


def _copy_block(table_ref, out_ref):
    out_ref[...] = table_ref[...][None]


def kernel(token_ids, table):
    T_max, C = table.shape
    _, T = token_ids.shape
    BLOCK = 2048
    grid = (T // BLOCK,)
    out = pl.pallas_call(
        _copy_block,
        grid=grid,
        in_specs=[pl.BlockSpec((BLOCK, C), lambda i: (i, 0))],
        out_specs=pl.BlockSpec((1, BLOCK, C), lambda i: (0, i, 0)),
        out_shape=jax.ShapeDtypeStruct((1, T, C), table.dtype),
    )(table)
    return out
